# baseline (speedup 1.0000x reference)
import functools

import jax
import jax.numpy as jnp
from jax.experimental import pallas as pl
from jax.experimental.pallas import tpu as pltpu


def _tc_broadcast_rows(pe2d, batch, length):
    d = pe2d.shape[1]
    chunk = 512
    nch = -(-length // chunk)
    lag = 3
    nbuf = 2 * lag + 1

    tail = length - (nch - 1) * chunk

    def body(pe_hbm, out_hbm, bufs, tbuf, gsems, wsems):
        def gcopy(i):
            return pltpu.make_async_copy(
                pe_hbm.at[pl.ds(i * chunk, chunk), :],
                bufs[i % nbuf], gsems[i % nbuf])

        def wcopy(i, b):
            if i == nch - 1:
                return pltpu.make_async_copy(
                    tbuf, out_hbm.at[b, pl.ds(i * chunk, tail), :],
                    wsems[i % nbuf])
            return pltpu.make_async_copy(
                bufs[i % nbuf],
                out_hbm.at[b, pl.ds(i * chunk, chunk), :],
                wsems[i % nbuf])

        for j in range(min(lag + 1, nch)):
            gcopy(j).start()
        for i in range(nch):
            gcopy(i).wait()
            if i == nch - 1:
                tbuf[...] = bufs[i % nbuf][pl.ds(0, tail), :]
            for b in range(batch):
                wcopy(i, b).start()
            if i >= lag:
                for b in range(batch):
                    wcopy(i - lag, b).wait()
            if i + lag + 1 < nch:
                gcopy(i + lag + 1).start()
        for j in range(max(nch - lag, 0), nch):
            for b in range(batch):
                wcopy(j, b).wait()

    return pl.pallas_call(
        body,
        in_specs=[pl.BlockSpec(memory_space=pl.ANY)],
        out_specs=pl.BlockSpec(memory_space=pl.ANY),
        out_shape=jax.ShapeDtypeStruct((batch, length, d), jnp.float32),
        scratch_shapes=[
            [pltpu.VMEM((chunk, d), jnp.float32) for _ in range(nbuf)],
            pltpu.VMEM((tail, d), jnp.float32),
            [pltpu.SemaphoreType.DMA for _ in range(nbuf)],
            [pltpu.SemaphoreType.DMA for _ in range(nbuf)],
        ],
    )(pe2d)


def kernel(x, pe):
    b, s, _ = x.shape
    length = 2 * s - 1
    return _tc_broadcast_rows(pe[0], b, length)


# speedup vs baseline: 1.0004x; 1.0004x over previous
import functools

import jax
import jax.numpy as jnp
from jax.experimental import pallas as pl
from jax.experimental.pallas import tpu as pltpu


def _tc_broadcast_rows(pe3d, batch, length):
    d = pe3d.shape[2]
    chunk = 512
    nch = -(-length // chunk)
    lag = 3
    nbuf = 2 * lag + 1

    tail = length - (nch - 1) * chunk

    def body(pe_hbm, out_hbm, bufs, tbuf, gsems, wsems):
        def gcopy(i):
            return pltpu.make_async_copy(
                pe_hbm.at[0, pl.ds(i * chunk, chunk), :],
                bufs[i % nbuf], gsems[i % nbuf])

        def wcopy(i, b):
            if i == nch - 1:
                return pltpu.make_async_copy(
                    tbuf, out_hbm.at[b, pl.ds(i * chunk, tail), :],
                    wsems[i % nbuf])
            return pltpu.make_async_copy(
                bufs[i % nbuf],
                out_hbm.at[b, pl.ds(i * chunk, chunk), :],
                wsems[i % nbuf])

        for j in range(min(lag + 1, nch)):
            gcopy(j).start()
        for i in range(nch):
            gcopy(i).wait()
            if i == nch - 1:
                tbuf[...] = bufs[i % nbuf][pl.ds(0, tail), :]
            for b in range(batch):
                wcopy(i, b).start()
            if i >= lag:
                for b in range(batch):
                    wcopy(i - lag, b).wait()
            if i + lag + 1 < nch:
                gcopy(i + lag + 1).start()
        for j in range(max(nch - lag, 0), nch):
            for b in range(batch):
                wcopy(j, b).wait()

    return pl.pallas_call(
        body,
        in_specs=[pl.BlockSpec(memory_space=pl.ANY)],
        out_specs=pl.BlockSpec(memory_space=pl.ANY),
        out_shape=jax.ShapeDtypeStruct((batch, length, d), jnp.float32),
        scratch_shapes=[
            [pltpu.VMEM((chunk, d), jnp.float32) for _ in range(nbuf)],
            pltpu.VMEM((tail, d), jnp.float32),
            [pltpu.SemaphoreType.DMA for _ in range(nbuf)],
            [pltpu.SemaphoreType.DMA for _ in range(nbuf)],
        ],
    )(pe3d)


def kernel(x, pe):
    b, s, _ = x.shape
    length = 2 * s - 1
    return _tc_broadcast_rows(pe, b, length)


# speedup vs baseline: 3.4522x; 3.4510x over previous
import functools

import jax
import jax.numpy as jnp
from jax.experimental import pallas as pl
from jax.experimental.pallas import tpu as pltpu


def _tc_broadcast_rows(pe3d, batch, length):
    d = pe3d.shape[2]
    chunk = 128
    nch = -(-length // chunk)
    lag = 3
    nbuf = 2 * lag + 1
    tail = length - (nch - 1) * chunk

    def body(pe_hbm, out_hbm, gbufs, wbufs, tbuf, gsems, wsems):
        def gcopy(i):
            return pltpu.make_async_copy(
                pe_hbm.at[0, pl.ds(i * chunk, chunk), :],
                gbufs[i % nbuf], gsems[i % nbuf])

        def wcopy(i):
            if i == nch - 1:
                return pltpu.make_async_copy(
                    tbuf, out_hbm.at[pl.ds(i * chunk, tail), :, :],
                    wsems[i % nbuf])
            return pltpu.make_async_copy(
                wbufs[i % nbuf],
                out_hbm.at[pl.ds(i * chunk, chunk), :, :],
                wsems[i % nbuf])

        for j in range(min(lag + 1, nch)):
            gcopy(j).start()
        for i in range(nch):
            gcopy(i).wait()
            if i == nch - 1:
                for b in range(batch):
                    tbuf[:, b, :] = gbufs[i % nbuf][pl.ds(0, tail), :]
            else:
                for b in range(batch):
                    wbufs[i % nbuf][:, b, :] = gbufs[i % nbuf][...]
            wcopy(i).start()
            if i >= lag:
                wcopy(i - lag).wait()
            if i + lag + 1 < nch:
                gcopy(i + lag + 1).start()
        for j in range(max(nch - lag, 0), nch):
            wcopy(j).wait()

    out = pl.pallas_call(
        body,
        in_specs=[pl.BlockSpec(memory_space=pl.ANY)],
        out_specs=pl.BlockSpec(memory_space=pl.ANY),
        out_shape=jax.ShapeDtypeStruct((length, batch, d), jnp.float32),
        scratch_shapes=[
            [pltpu.VMEM((chunk, d), jnp.float32) for _ in range(nbuf)],
            [pltpu.VMEM((chunk, batch, d), jnp.float32) for _ in range(nbuf)],
            pltpu.VMEM((tail, batch, d), jnp.float32),
            [pltpu.SemaphoreType.DMA for _ in range(nbuf)],
            [pltpu.SemaphoreType.DMA for _ in range(nbuf)],
        ],
    )(pe3d)
    return jnp.transpose(out, (1, 0, 2))


def kernel(x, pe):
    b, s, _ = x.shape
    length = 2 * s - 1
    return _tc_broadcast_rows(pe, b, length)


# speedup vs baseline: 3.4679x; 1.0045x over previous
import functools

import jax
import jax.numpy as jnp
from jax.experimental import pallas as pl
from jax.experimental.pallas import tpu as pltpu


def _tc_broadcast_rows(pe3d, batch, length):
    d = pe3d.shape[2]
    chunk = 128
    nch = -(-length // chunk)
    lag = 4
    nbuf = 2 * lag + 1
    tail = length - (nch - 1) * chunk

    def body(pe_hbm, out_hbm, gbufs, wbufs, tbuf, gsems, wsems):
        def gcopy(i):
            return pltpu.make_async_copy(
                pe_hbm.at[0, pl.ds(i * chunk, chunk), :],
                gbufs[i % nbuf], gsems[i % nbuf])

        def wcopy(i):
            if i == nch - 1:
                return pltpu.make_async_copy(
                    tbuf, out_hbm.at[pl.ds(i * chunk, tail), :, :],
                    wsems[i % nbuf])
            return pltpu.make_async_copy(
                wbufs[i % nbuf],
                out_hbm.at[pl.ds(i * chunk, chunk), :, :],
                wsems[i % nbuf])

        for j in range(min(lag + 1, nch)):
            gcopy(j).start()
        for i in range(nch):
            gcopy(i).wait()
            if i == nch - 1:
                for b in range(batch):
                    tbuf[:, b, :] = gbufs[i % nbuf][pl.ds(0, tail), :]
            else:
                for b in range(batch):
                    wbufs[i % nbuf][:, b, :] = gbufs[i % nbuf][...]
            wcopy(i).start()
            if i >= lag:
                wcopy(i - lag).wait()
            if i + lag + 1 < nch:
                gcopy(i + lag + 1).start()
        for j in range(max(nch - lag, 0), nch):
            wcopy(j).wait()

    out = pl.pallas_call(
        body,
        in_specs=[pl.BlockSpec(memory_space=pl.ANY)],
        out_specs=pl.BlockSpec(memory_space=pl.ANY),
        out_shape=jax.ShapeDtypeStruct((length, batch, d), jnp.float32),
        scratch_shapes=[
            [pltpu.VMEM((chunk, d), jnp.float32) for _ in range(nbuf)],
            [pltpu.VMEM((chunk, batch, d), jnp.float32) for _ in range(nbuf)],
            pltpu.VMEM((tail, batch, d), jnp.float32),
            [pltpu.SemaphoreType.DMA for _ in range(nbuf)],
            [pltpu.SemaphoreType.DMA for _ in range(nbuf)],
        ],
    )(pe3d)
    return jnp.transpose(out, (1, 0, 2))


def kernel(x, pe):
    b, s, _ = x.shape
    length = 2 * s - 1
    return _tc_broadcast_rows(pe, b, length)
